# baseline (speedup 1.0000x reference)
import functools

import jax
import jax.numpy as jnp
from jax import lax
from jax.experimental import pallas as pl
from jax.experimental.pallas import tpu as pltpu
from jax.experimental.pallas import tpu_sc as plsc

N = 50000
E = 800000
IN = 128
H = 96
OUT = 128

CW = 32
NCHUNK = H // CW
NC = 2
NS = 16

G = 3
NGROUP = 132
H2 = NGROUP // 2
ROWS_PT = G * NGROUP
ROWS_PAD = NS * ROWS_PT
E_PAD = ROWS_PAD * 128

SPLIT_DEG = 52
SPLIT_NODEG = 46

NP = 50048
RPT_N = NP // NS

BLK = 1000
NBLK = N // BLK


def _ln(x, g, b, eps=1e-5):
    mu = jnp.mean(x, axis=-1, keepdims=True)
    d = x - mu
    var = jnp.mean(d * d, axis=-1, keepdims=True)
    return d / jnp.sqrt(var + eps) * g + b


def _tc_in_body(x_ref, W0_ref, b0_ref, g0_ref, be0_ref, W1_ref, b1_ref,
                g1_ref, be1_ref, h_ref, *chunk_refs):
    x = x_ref[...]
    h = jnp.maximum(jnp.dot(x, W0_ref[...], preferred_element_type=jnp.float32)
                    + b0_ref[...], 0.0)
    h = _ln(h, g0_ref[...], be0_ref[...])
    h = jnp.maximum(jnp.dot(h, W1_ref[...], preferred_element_type=jnp.float32)
                    + b1_ref[...], 0.0)
    h = _ln(h, g1_ref[...], be1_ref[...])
    h_ref[...] = h
    for c in range(NCHUNK):
        chunk_refs[c][...] = h[:, c * CW:(c + 1) * CW]


def _tc_in(x, W0, b0, g0, be0, W1, b1, g1, be1):
    full = lambda s: pl.BlockSpec(s, lambda i: (0, 0))
    return pl.pallas_call(
        _tc_in_body,
        grid=(NBLK,),
        in_specs=[
            pl.BlockSpec((BLK, IN), lambda i: (i, 0)),
            full((IN, H)), full((1, H)), full((1, H)), full((1, H)),
            full((H, H)), full((1, H)), full((1, H)), full((1, H)),
        ],
        out_specs=[pl.BlockSpec((BLK, H), lambda i: (i, 0))]
        + [pl.BlockSpec((BLK, CW), lambda i: (i, 0))] * NCHUNK,
        out_shape=[jax.ShapeDtypeStruct((N, H), jnp.float32)]
        + [jax.ShapeDtypeStruct((N, CW), jnp.float32)] * NCHUNK,
    )(x, W0, b0, g0, be0, W1, b1, g1, be1)


def _make_sc(with_deg):
    mesh = plsc.VectorSubcoreMesh(core_axis_name="c", subcore_axis_name="s",
                                  num_cores=NC, num_subcores=NS)
    n_out = 4 + (2 if with_deg else 0)
    out_type = [jax.ShapeDtypeStruct((NP, CW), jnp.float32)] * n_out
    RB = G * 128
    scratch = [
        pltpu.VMEM((G, 128), jnp.int32),
        pltpu.VMEM((G, 128), jnp.int32),
        pltpu.VMEM((G, 128), jnp.int32),
        pltpu.VMEM((G, 128), jnp.int32),
        pltpu.VMEM((RB, CW), jnp.float32),
        pltpu.VMEM((RB, CW), jnp.float32),
        pltpu.VMEM_SHARED((NP, CW), jnp.float32),
        pltpu.SemaphoreType.DMA,
        pltpu.SemaphoreType.DMA,
        pltpu.SemaphoreType.DMA,
        pltpu.SemaphoreType.DMA,
        pltpu.SemaphoreType.DMA,
        pltpu.SemaphoreType.DMA,
    ]

    @functools.partial(
        pl.kernel, out_type=out_type, mesh=mesh, scratch_types=scratch,
        compiler_params=pltpu.CompilerParams(use_tc_tiling_on_sc=False))
    def sc_kernel(src_hbm, dst_hbm, *rest):
        tables = rest[:NCHUNK]
        outs = rest[NCHUNK:NCHUNK + n_out]
        (src_sl0, src_sl1, dst_sl0, dst_sl1, rows0, rows1, acc,
         gsem, ssem0, ssem1, isrc, idst, msem) = rest[NCHUNK + n_out:]
        src_sl = (src_sl0, src_sl1)
        dst_sl = (dst_sl0, dst_sl1)
        rows = (rows0, rows1)
        ssems = (ssem0, ssem1)
        cid = lax.axis_index("c")
        sid = lax.axis_index("s")
        row0 = sid * ROWS_PT
        nbase = sid * RPT_N

        def fill(buf, val):
            def body(i, carry):
                for half in range(CW // 16):
                    buf[i, pl.ds(half * 16, 16)] = jnp.full(
                        (16,), val, jnp.float32)
                return carry
            lax.fori_loop(0, 128, body, 0)

        def zero_slice(dst):
            ds_ = [pltpu.async_copy(
                rows1.at[pl.ds(0, 128)], dst.at[pl.ds(nbase + k * 128, 128)],
                msem)
                for k in range(RPT_N // 128)]
            ds_.append(pltpu.async_copy(
                rows1.at[pl.ds(0, RPT_N % 128)],
                dst.at[pl.ds(nbase + (RPT_N // 128) * 128, RPT_N % 128)],
                msem))
            for d in ds_:
                d.wait()

        def fire_gathers(table, buf):
            for r in range(G):
                pltpu.async_copy(table.at[src_sl[buf].at[r]],
                                 rows[buf].at[pl.ds(r * 128, 128)], gsem)

        def wait_gathers(table, buf):
            for r in range(G):
                pltpu.make_async_copy(table.at[src_sl[buf].at[r]],
                                      rows[buf].at[pl.ds(r * 128, 128)],
                                      gsem).wait()

        def fire_scatters(buf):
            for r in range(G):
                pltpu.async_copy(rows[buf].at[pl.ds(r * 128, 128)],
                                 acc.at[dst_sl[buf].at[r]], ssems[buf],
                                 add=True)

        def wait_scatters(buf):
            for r in range(G):
                pltpu.make_async_copy(rows[buf].at[pl.ds(r * 128, 128)],
                                      acc.at[dst_sl[buf].at[r]],
                                      ssems[buf]).wait()

        def fire_deg(buf):
            for r in range(G):
                pltpu.async_copy(rows0.at[pl.ds(0, 128)],
                                 acc.at[dst_sl[buf].at[r]], ssems[buf],
                                 add=True)

        def wait_deg(buf):
            for r in range(G):
                pltpu.make_async_copy(rows0.at[pl.ds(0, 128)],
                                      acc.at[dst_sl[buf].at[r]],
                                      ssems[buf]).wait()

        def fire_load(hbm, g, slot, sem):
            pltpu.async_copy(hbm.at[pl.ds(row0 + g * G, G)], slot, sem)

        def wait_load(hbm, slot, sem):
            pltpu.make_async_copy(hbm.at[pl.ds(row0, G)], slot, sem).wait()

        def finish_pass(out):
            plsc.subcore_barrier()
            pltpu.sync_copy(acc.at[pl.ds(nbase, RPT_N)],
                            out.at[pl.ds(nbase, RPT_N)])
            plsc.subcore_barrier()

        def do_pass(table, out, g0, ng):
            pltpu.sync_copy(src_hbm.at[pl.ds(row0 + g0 * G, G)], src_sl[0])
            pltpu.sync_copy(src_hbm.at[pl.ds(row0 + (g0 + 1) * G, G)],
                            src_sl[1])
            pltpu.sync_copy(dst_hbm.at[pl.ds(row0 + g0 * G, G)], dst_sl[0])
            pltpu.sync_copy(dst_hbm.at[pl.ds(row0 + (g0 + 1) * G, G)],
                            dst_sl[1])
            fill(rows1, 0.0)
            zero_slice(acc)
            plsc.subcore_barrier()

            fire_gathers(table, 0)

            def halfstep(i, carry):
                for b in (0, 1):
                    g = 2 * i + b
                    wait_gathers(table, b)

                    @pl.when(g + 2 < ng)
                    def _():
                        fire_load(src_hbm, g0 + g + 2, src_sl[b], isrc)

                    @pl.when(g >= 2)
                    def _():
                        wait_load(dst_hbm, dst_sl[b], idst)

                    fire_scatters(b)

                    @pl.when(g > 0)
                    def _():
                        wait_scatters(1 - b)

                    @pl.when(jnp.logical_and(g >= 1, g + 1 < ng))
                    def _():
                        fire_load(dst_hbm, g0 + g + 1, dst_sl[1 - b], idst)

                    @pl.when(jnp.logical_and(g >= 1, g + 1 < ng))
                    def _():
                        wait_load(src_hbm, src_sl[1 - b], isrc)

                    @pl.when(g + 1 < ng)
                    def _():
                        fire_gathers(table, 1 - b)
                return carry
            lax.fori_loop(0, ng // 2, halfstep, 0)
            wait_scatters(1)
            finish_pass(out)

        def do_deg_pass(out, g0, ng):
            pltpu.sync_copy(dst_hbm.at[pl.ds(row0 + g0 * G, G)], dst_sl[0])
            pltpu.sync_copy(dst_hbm.at[pl.ds(row0 + (g0 + 1) * G, G)],
                            dst_sl[1])
            fill(rows0, 1.0)
            fill(rows1, 0.0)
            zero_slice(acc)
            plsc.subcore_barrier()

            def halfstep(i, carry):
                for b in (0, 1):
                    g = 2 * i + b

                    @pl.when(g >= 2)
                    def _():
                        wait_load(dst_hbm, dst_sl[b], idst)

                    fire_deg(b)

                    @pl.when(g > 0)
                    def _():
                        wait_deg(1 - b)

                    @pl.when(jnp.logical_and(g >= 1, g + 1 < ng))
                    def _():
                        fire_load(dst_hbm, g0 + g + 1, dst_sl[1 - b], idst)
                return carry
            lax.fori_loop(0, ng // 2, halfstep, 0)
            wait_deg(1)
            finish_pass(out)

        S = SPLIT_DEG if with_deg else SPLIT_NODEG

        @pl.when(cid == 0)
        def _():
            do_pass(tables[0], outs[0], 0, NGROUP)
            do_pass(tables[2], outs[2], 0, S)
            if with_deg:
                do_deg_pass(outs[4], 0, S)

        @pl.when(cid == 1)
        def _():
            do_pass(tables[1], outs[1], 0, NGROUP)
            do_pass(tables[2], outs[3], S, NGROUP - S)
            if with_deg:
                do_deg_pass(outs[5], S, NGROUP - S)

    return sc_kernel


@functools.lru_cache(maxsize=None)
def _get_sc(with_deg):
    return _make_sc(with_deg)


def _mix_common(x, m_chunks, deg, Wx, Wm, b, tm):
    m = jnp.concatenate(m_chunks, axis=1)
    deg = jnp.maximum(deg, 1.0)
    m = m / deg
    z = (jnp.dot(x, Wx, preferred_element_type=jnp.float32)
         + jnp.dot(m, Wm, preferred_element_type=jnp.float32) + b)
    z = z - jnp.max(z, axis=-1, keepdims=True)
    ez = jnp.exp(z)
    raw = ez / jnp.sum(ez, axis=-1, keepdims=True)
    r_i = lax.broadcasted_iota(jnp.int32, (H, H), 0)
    c_i = lax.broadcasted_iota(jnp.int32, (H, H), 1)
    tri = (r_i <= c_i).astype(jnp.float32)
    cum = jnp.dot(raw, tri, preferred_element_type=jnp.float32)
    gate = tm + (1.0 - tm) * cum
    out = x * gate + m * (1.0 - gate)
    return out, gate, m


def _tc_mix1_body(x_ref, m0, m1, m2a, m2b, dega_ref, degb_ref, Wx_ref, Wm_ref,
                  b_ref, g_ref, be_ref, x2_ref, gate_ref, *chunk_refs):
    deg = dega_ref[...][:, :1] + degb_ref[...][:, :1]
    out, gate, _ = _mix_common(
        x_ref[...], [m0[...], m1[...], m2a[...] + m2b[...]], deg,
        Wx_ref[...], Wm_ref[...], b_ref[...], 0.0)
    x2 = _ln(out, g_ref[...], be_ref[...])
    x2_ref[...] = x2
    gate_ref[...] = gate
    for c in range(NCHUNK):
        chunk_refs[c][...] = x2[:, c * CW:(c + 1) * CW]


def _tc_mix1(x, m_chunks, dega, degb, Wx, Wm, b, g, be):
    full = lambda s: pl.BlockSpec(s, lambda i: (0, 0))
    row = lambda w: pl.BlockSpec((BLK, w), lambda i: (i, 0))
    return pl.pallas_call(
        _tc_mix1_body,
        grid=(NBLK,),
        in_specs=[row(H)] + [row(CW)] * 6
        + [full((H, H)), full((H, H)), full((1, H)), full((1, H)),
           full((1, H))],
        out_specs=[row(H), row(H)] + [row(CW)] * NCHUNK,
        out_shape=[jax.ShapeDtypeStruct((N, H), jnp.float32)] * 2
        + [jax.ShapeDtypeStruct((N, CW), jnp.float32)] * NCHUNK,
    )(x, *m_chunks, dega, degb, Wx, Wm, b, g, be)


def _tc_mix2_body(x_ref, m0, m1, m2a, m2b, dega_ref, degb_ref, tm_ref, Wx_ref,
                  Wm_ref, b_ref, g_ref, be_ref, Wo_ref, bo_ref, y_ref):
    deg = dega_ref[...][:, :1] + degb_ref[...][:, :1]
    out, _, _ = _mix_common(
        x_ref[...], [m0[...], m1[...], m2a[...] + m2b[...]], deg,
        Wx_ref[...], Wm_ref[...], b_ref[...], tm_ref[...])
    x3 = _ln(out, g_ref[...], be_ref[...])
    y_ref[...] = (jnp.dot(x3, Wo_ref[...], preferred_element_type=jnp.float32)
                  + bo_ref[...])


def _tc_mix2(x, m_chunks, dega, degb, tm, Wx, Wm, b, g, be, Wo, bo):
    full = lambda s: pl.BlockSpec(s, lambda i: (0, 0))
    row = lambda w: pl.BlockSpec((BLK, w), lambda i: (i, 0))
    return pl.pallas_call(
        _tc_mix2_body,
        grid=(NBLK,),
        in_specs=[row(H)] + [row(CW)] * 6 + [row(H)]
        + [full((H, H)), full((H, H)), full((1, H)), full((1, H)),
           full((1, H)), full((H, OUT)), full((1, OUT))],
        out_specs=pl.BlockSpec((BLK, OUT), lambda i: (i, 0)),
        out_shape=jax.ShapeDtypeStruct((N, OUT), jnp.float32),
    )(x, *m_chunks, dega, degb, tm, Wx, Wm, b, g, be, Wo, bo)


def kernel(x, edge_index, W_in0, b_in0, g_in0, be_in0, W_in1, b_in1, g_in1,
           be_in1, W_tm0, b_tm0, g_tm0, be_tm0, W_tm1, b_tm1, g_tm1, be_tm1,
           W_out, b_out):
    r1 = lambda v: v.reshape(1, -1)
    pad = E_PAD - E
    src_p = jnp.concatenate(
        [edge_index[0], jnp.zeros((pad,), jnp.int32)]).reshape(ROWS_PAD, 128)
    dst_p = jnp.concatenate(
        [edge_index[1], jnp.full((pad,), N, jnp.int32)]).reshape(ROWS_PAD, 128)

    outs = _tc_in(x, W_in0, r1(b_in0), r1(g_in0), r1(be_in0),
                  W_in1, r1(b_in1), r1(g_in1), r1(be_in1))
    h, h_chunks = outs[0], outs[1:]

    sc1 = _get_sc(True)(src_p, dst_p, *h_chunks)
    m1_chunks = sc1[:4]
    dega, degb = sc1[4], sc1[5]

    outs = _tc_mix1(h, m1_chunks, dega, degb, W_tm0[:H], W_tm0[H:],
                    r1(b_tm0), r1(g_tm0), r1(be_tm0))
    x2, gate1, x2_chunks = outs[0], outs[1], outs[2:]

    m2_chunks = _get_sc(False)(src_p, dst_p, *x2_chunks)

    return _tc_mix2(x2, m2_chunks, dega, degb, gate1, W_tm1[:H], W_tm1[H:],
                    r1(b_tm1), r1(g_tm1), r1(be_tm1), W_out, r1(b_out))


# speedup vs baseline: 1.0434x; 1.0434x over previous
import functools

import jax
import jax.numpy as jnp
from jax import lax
from jax.experimental import pallas as pl
from jax.experimental.pallas import tpu as pltpu
from jax.experimental.pallas import tpu_sc as plsc

N = 50000
E = 800000
IN = 128
H = 96
OUT = 128

CW = 32
NCHUNK = H // CW
NC = 2
NS = 16

G = 3
NGROUP = 132
H2 = NGROUP // 2
ROWS_PT = G * NGROUP
ROWS_PAD = NS * ROWS_PT
E_PAD = ROWS_PAD * 128

SPLIT_DEG = 80
SPLIT_NODEG = 86

NP = 50048
RPT_N = NP // NS

BLK = 1000
NBLK = N // BLK


def _ln(x, g, b, eps=1e-5):
    mu = jnp.mean(x, axis=-1, keepdims=True)
    d = x - mu
    var = jnp.mean(d * d, axis=-1, keepdims=True)
    return d / jnp.sqrt(var + eps) * g + b


def _tc_in_body(x_ref, W0_ref, b0_ref, g0_ref, be0_ref, W1_ref, b1_ref,
                g1_ref, be1_ref, h_ref, *chunk_refs):
    x = x_ref[...]
    h = jnp.maximum(jnp.dot(x, W0_ref[...], preferred_element_type=jnp.float32)
                    + b0_ref[...], 0.0)
    h = _ln(h, g0_ref[...], be0_ref[...])
    h = jnp.maximum(jnp.dot(h, W1_ref[...], preferred_element_type=jnp.float32)
                    + b1_ref[...], 0.0)
    h = _ln(h, g1_ref[...], be1_ref[...])
    h_ref[...] = h
    for c in range(NCHUNK):
        chunk_refs[c][...] = h[:, c * CW:(c + 1) * CW]


def _tc_in(x, W0, b0, g0, be0, W1, b1, g1, be1):
    full = lambda s: pl.BlockSpec(s, lambda i: (0, 0))
    return pl.pallas_call(
        _tc_in_body,
        grid=(NBLK,),
        in_specs=[
            pl.BlockSpec((BLK, IN), lambda i: (i, 0)),
            full((IN, H)), full((1, H)), full((1, H)), full((1, H)),
            full((H, H)), full((1, H)), full((1, H)), full((1, H)),
        ],
        out_specs=[pl.BlockSpec((BLK, H), lambda i: (i, 0))]
        + [pl.BlockSpec((BLK, CW), lambda i: (i, 0))] * NCHUNK,
        out_shape=[jax.ShapeDtypeStruct((N, H), jnp.float32)]
        + [jax.ShapeDtypeStruct((N, CW), jnp.float32)] * NCHUNK,
    )(x, W0, b0, g0, be0, W1, b1, g1, be1)


def _make_sc(with_deg):
    mesh = plsc.VectorSubcoreMesh(core_axis_name="c", subcore_axis_name="s",
                                  num_cores=NC, num_subcores=NS)
    n_out = 4 + (2 if with_deg else 0)
    out_type = [jax.ShapeDtypeStruct((NP, CW), jnp.float32)] * n_out
    RB = G * 128
    scratch = [
        pltpu.VMEM((G, 128), jnp.int32),
        pltpu.VMEM((G, 128), jnp.int32),
        pltpu.VMEM((G, 128), jnp.int32),
        pltpu.VMEM((G, 128), jnp.int32),
        pltpu.VMEM((RB, CW), jnp.float32),
        pltpu.VMEM((RB, CW), jnp.float32),
        pltpu.VMEM_SHARED((NP, CW), jnp.float32),
        pltpu.SemaphoreType.DMA,
        pltpu.SemaphoreType.DMA,
        pltpu.SemaphoreType.DMA,
        pltpu.SemaphoreType.DMA,
        pltpu.SemaphoreType.DMA,
        pltpu.SemaphoreType.DMA,
    ]

    @functools.partial(
        pl.kernel, out_type=out_type, mesh=mesh, scratch_types=scratch,
        compiler_params=pltpu.CompilerParams(use_tc_tiling_on_sc=False))
    def sc_kernel(src_hbm, dst_hbm, *rest):
        tables = rest[:NCHUNK]
        outs = rest[NCHUNK:NCHUNK + n_out]
        (src_sl0, src_sl1, dst_sl0, dst_sl1, rows0, rows1, acc,
         gsem, ssem0, ssem1, isrc, idst, msem) = rest[NCHUNK + n_out:]
        src_sl = (src_sl0, src_sl1)
        dst_sl = (dst_sl0, dst_sl1)
        rows = (rows0, rows1)
        ssems = (ssem0, ssem1)
        cid = lax.axis_index("c")
        sid = lax.axis_index("s")
        row0 = sid * ROWS_PT
        nbase = sid * RPT_N

        def fill(buf, val):
            def body(i, carry):
                for half in range(CW // 16):
                    buf[i, pl.ds(half * 16, 16)] = jnp.full(
                        (16,), val, jnp.float32)
                return carry
            lax.fori_loop(0, 128, body, 0)

        def zero_slice(dst):
            ds_ = [pltpu.async_copy(
                rows1.at[pl.ds(0, 128)], dst.at[pl.ds(nbase + k * 128, 128)],
                msem)
                for k in range(RPT_N // 128)]
            ds_.append(pltpu.async_copy(
                rows1.at[pl.ds(0, RPT_N % 128)],
                dst.at[pl.ds(nbase + (RPT_N // 128) * 128, RPT_N % 128)],
                msem))
            for d in ds_:
                d.wait()

        def fire_gathers(table, buf):
            for r in range(G):
                pltpu.async_copy(table.at[src_sl[buf].at[r]],
                                 rows[buf].at[pl.ds(r * 128, 128)], gsem)

        def wait_gathers(table, buf):
            for r in range(G):
                pltpu.make_async_copy(table.at[src_sl[buf].at[r]],
                                      rows[buf].at[pl.ds(r * 128, 128)],
                                      gsem).wait()

        def fire_scatters(buf):
            for r in range(G):
                pltpu.async_copy(rows[buf].at[pl.ds(r * 128, 128)],
                                 acc.at[dst_sl[buf].at[r]], ssems[buf],
                                 add=True)

        def wait_scatters(buf):
            for r in range(G):
                pltpu.make_async_copy(rows[buf].at[pl.ds(r * 128, 128)],
                                      acc.at[dst_sl[buf].at[r]],
                                      ssems[buf]).wait()

        def fire_deg(buf):
            for r in range(G):
                pltpu.async_copy(rows0.at[pl.ds(0, 128)],
                                 acc.at[dst_sl[buf].at[r]], ssems[buf],
                                 add=True)

        def wait_deg(buf):
            for r in range(G):
                pltpu.make_async_copy(rows0.at[pl.ds(0, 128)],
                                      acc.at[dst_sl[buf].at[r]],
                                      ssems[buf]).wait()

        def fire_load(hbm, g, slot, sem):
            pltpu.async_copy(hbm.at[pl.ds(row0 + g * G, G)], slot, sem)

        def wait_load(hbm, slot, sem):
            pltpu.make_async_copy(hbm.at[pl.ds(row0, G)], slot, sem).wait()

        def finish_pass(out):
            plsc.subcore_barrier()
            pltpu.sync_copy(acc.at[pl.ds(nbase, RPT_N)],
                            out.at[pl.ds(nbase, RPT_N)])
            plsc.subcore_barrier()

        def do_pass(table, out, g0, ng):
            pltpu.sync_copy(src_hbm.at[pl.ds(row0 + g0 * G, G)], src_sl[0])
            pltpu.sync_copy(src_hbm.at[pl.ds(row0 + (g0 + 1) * G, G)],
                            src_sl[1])
            pltpu.sync_copy(dst_hbm.at[pl.ds(row0 + g0 * G, G)], dst_sl[0])
            pltpu.sync_copy(dst_hbm.at[pl.ds(row0 + (g0 + 1) * G, G)],
                            dst_sl[1])
            fill(rows1, 0.0)
            zero_slice(acc)
            plsc.subcore_barrier()

            fire_gathers(table, 0)

            def halfstep(i, carry):
                for b in (0, 1):
                    g = 2 * i + b
                    wait_gathers(table, b)

                    @pl.when(g + 2 < ng)
                    def _():
                        fire_load(src_hbm, g0 + g + 2, src_sl[b], isrc)

                    @pl.when(g >= 2)
                    def _():
                        wait_load(dst_hbm, dst_sl[b], idst)

                    fire_scatters(b)

                    @pl.when(g > 0)
                    def _():
                        wait_scatters(1 - b)

                    @pl.when(jnp.logical_and(g >= 1, g + 1 < ng))
                    def _():
                        fire_load(dst_hbm, g0 + g + 1, dst_sl[1 - b], idst)

                    @pl.when(jnp.logical_and(g >= 1, g + 1 < ng))
                    def _():
                        wait_load(src_hbm, src_sl[1 - b], isrc)

                    @pl.when(g + 1 < ng)
                    def _():
                        fire_gathers(table, 1 - b)
                return carry
            lax.fori_loop(0, ng // 2, halfstep, 0)
            wait_scatters(1)
            finish_pass(out)

        def do_deg_pass(out, g0, ng):
            pltpu.sync_copy(dst_hbm.at[pl.ds(row0 + g0 * G, G)], dst_sl[0])
            pltpu.sync_copy(dst_hbm.at[pl.ds(row0 + (g0 + 1) * G, G)],
                            dst_sl[1])
            fill(rows0, 1.0)
            fill(rows1, 0.0)
            zero_slice(acc)
            plsc.subcore_barrier()

            def halfstep(i, carry):
                for b in (0, 1):
                    g = 2 * i + b

                    @pl.when(g >= 2)
                    def _():
                        wait_load(dst_hbm, dst_sl[b], idst)

                    fire_deg(b)

                    @pl.when(g > 0)
                    def _():
                        wait_deg(1 - b)

                    @pl.when(jnp.logical_and(g >= 1, g + 1 < ng))
                    def _():
                        fire_load(dst_hbm, g0 + g + 1, dst_sl[1 - b], idst)
                return carry
            lax.fori_loop(0, ng // 2, halfstep, 0)
            wait_deg(1)
            finish_pass(out)

        S = SPLIT_DEG if with_deg else SPLIT_NODEG

        @pl.when(cid == 0)
        def _():
            do_pass(tables[0], outs[0], 0, NGROUP)
            do_pass(tables[2], outs[2], 0, S)
            if with_deg:
                do_deg_pass(outs[4], 0, S)

        @pl.when(cid == 1)
        def _():
            do_pass(tables[1], outs[1], 0, NGROUP)
            do_pass(tables[2], outs[3], S, NGROUP - S)
            if with_deg:
                do_deg_pass(outs[5], S, NGROUP - S)

    return sc_kernel


@functools.lru_cache(maxsize=None)
def _get_sc(with_deg):
    return _make_sc(with_deg)


def _mix_common(x, m_chunks, deg, Wx, Wm, b, tm):
    m = jnp.concatenate(m_chunks, axis=1)
    deg = jnp.maximum(deg, 1.0)
    m = m / deg
    z = (jnp.dot(x, Wx, preferred_element_type=jnp.float32)
         + jnp.dot(m, Wm, preferred_element_type=jnp.float32) + b)
    z = z - jnp.max(z, axis=-1, keepdims=True)
    ez = jnp.exp(z)
    raw = ez / jnp.sum(ez, axis=-1, keepdims=True)
    r_i = lax.broadcasted_iota(jnp.int32, (H, H), 0)
    c_i = lax.broadcasted_iota(jnp.int32, (H, H), 1)
    tri = (r_i <= c_i).astype(jnp.float32)
    cum = jnp.dot(raw, tri, preferred_element_type=jnp.float32)
    gate = tm + (1.0 - tm) * cum
    out = x * gate + m * (1.0 - gate)
    return out, gate, m


def _tc_mix1_body(x_ref, m0, m1, m2a, m2b, dega_ref, degb_ref, Wx_ref, Wm_ref,
                  b_ref, g_ref, be_ref, x2_ref, gate_ref, *chunk_refs):
    deg = dega_ref[...][:, :1] + degb_ref[...][:, :1]
    out, gate, _ = _mix_common(
        x_ref[...], [m0[...], m1[...], m2a[...] + m2b[...]], deg,
        Wx_ref[...], Wm_ref[...], b_ref[...], 0.0)
    x2 = _ln(out, g_ref[...], be_ref[...])
    x2_ref[...] = x2
    gate_ref[...] = gate
    for c in range(NCHUNK):
        chunk_refs[c][...] = x2[:, c * CW:(c + 1) * CW]


def _tc_mix1(x, m_chunks, dega, degb, Wx, Wm, b, g, be):
    full = lambda s: pl.BlockSpec(s, lambda i: (0, 0))
    row = lambda w: pl.BlockSpec((BLK, w), lambda i: (i, 0))
    return pl.pallas_call(
        _tc_mix1_body,
        grid=(NBLK,),
        in_specs=[row(H)] + [row(CW)] * 6
        + [full((H, H)), full((H, H)), full((1, H)), full((1, H)),
           full((1, H))],
        out_specs=[row(H), row(H)] + [row(CW)] * NCHUNK,
        out_shape=[jax.ShapeDtypeStruct((N, H), jnp.float32)] * 2
        + [jax.ShapeDtypeStruct((N, CW), jnp.float32)] * NCHUNK,
    )(x, *m_chunks, dega, degb, Wx, Wm, b, g, be)


def _tc_mix2_body(x_ref, m0, m1, m2a, m2b, dega_ref, degb_ref, tm_ref, Wx_ref,
                  Wm_ref, b_ref, g_ref, be_ref, Wo_ref, bo_ref, y_ref):
    deg = dega_ref[...][:, :1] + degb_ref[...][:, :1]
    out, _, _ = _mix_common(
        x_ref[...], [m0[...], m1[...], m2a[...] + m2b[...]], deg,
        Wx_ref[...], Wm_ref[...], b_ref[...], tm_ref[...])
    x3 = _ln(out, g_ref[...], be_ref[...])
    y_ref[...] = (jnp.dot(x3, Wo_ref[...], preferred_element_type=jnp.float32)
                  + bo_ref[...])


def _tc_mix2(x, m_chunks, dega, degb, tm, Wx, Wm, b, g, be, Wo, bo):
    full = lambda s: pl.BlockSpec(s, lambda i: (0, 0))
    row = lambda w: pl.BlockSpec((BLK, w), lambda i: (i, 0))
    return pl.pallas_call(
        _tc_mix2_body,
        grid=(NBLK,),
        in_specs=[row(H)] + [row(CW)] * 6 + [row(H)]
        + [full((H, H)), full((H, H)), full((1, H)), full((1, H)),
           full((1, H)), full((H, OUT)), full((1, OUT))],
        out_specs=pl.BlockSpec((BLK, OUT), lambda i: (i, 0)),
        out_shape=jax.ShapeDtypeStruct((N, OUT), jnp.float32),
    )(x, *m_chunks, dega, degb, tm, Wx, Wm, b, g, be, Wo, bo)


def kernel(x, edge_index, W_in0, b_in0, g_in0, be_in0, W_in1, b_in1, g_in1,
           be_in1, W_tm0, b_tm0, g_tm0, be_tm0, W_tm1, b_tm1, g_tm1, be_tm1,
           W_out, b_out):
    r1 = lambda v: v.reshape(1, -1)
    pad = E_PAD - E
    src_p = jnp.concatenate(
        [edge_index[0], jnp.zeros((pad,), jnp.int32)]).reshape(ROWS_PAD, 128)
    dst_p = jnp.concatenate(
        [edge_index[1], jnp.full((pad,), N, jnp.int32)]).reshape(ROWS_PAD, 128)

    outs = _tc_in(x, W_in0, r1(b_in0), r1(g_in0), r1(be_in0),
                  W_in1, r1(b_in1), r1(g_in1), r1(be_in1))
    h, h_chunks = outs[0], outs[1:]

    sc1 = _get_sc(True)(src_p, dst_p, *h_chunks)
    m1_chunks = sc1[:4]
    dega, degb = sc1[4], sc1[5]

    outs = _tc_mix1(h, m1_chunks, dega, degb, W_tm0[:H], W_tm0[H:],
                    r1(b_tm0), r1(g_tm0), r1(be_tm0))
    x2, gate1, x2_chunks = outs[0], outs[1], outs[2:]

    m2_chunks = _get_sc(False)(src_p, dst_p, *x2_chunks)

    return _tc_mix2(x2, m2_chunks, dega, degb, gate1, W_tm1[:H], W_tm1[H:],
                    r1(b_tm1), r1(g_tm1), r1(be_tm1), W_out, r1(b_out))
